# Initial kernel scaffold; baseline (speedup 1.0000x reference)
import jax
import jax.numpy as jnp
from jax.experimental import pallas as pl


def kernel(atomic_positions, grid_points, W1, b1, W2, b2, W3, b3, Wn1, bn1, Wn2, bn2, Wh1, bh1, Wh2, bh2):
    raise NotImplementedError("write your pallas kernel here")


# speedup vs baseline: 1.4676x; 1.4676x over previous
import jax
import jax.numpy as jnp
from jax.experimental import pallas as pl
from jax.experimental.pallas import tpu as pltpu

N = 4096
M = 8192
H = 256
BN = 512
NB = N // BN
CM = 2048
NC = M // CM
BIG = 2 ** 30


def _bdot(a, b):
    return jnp.dot(a.astype(jnp.bfloat16), b.astype(jnp.bfloat16),
                   preferred_element_type=jnp.float32)


def _enc_argmin_kernel(posP_ref, gridP_ref, W1_ref, b1_ref, W2_ref, b2_ref,
                       W3_ref, b3_ref, feats_ref, idx_ref, gB_ref, gsq_ref):
    b = pl.program_id(0)

    @pl.when(b == 0)
    def _build_grid_aug():
        g = gridP_ref[...]
        gB_ref[...] = g.astype(jnp.bfloat16)
        gsq_ref[...] = ((g[0:1] * g[0:1] + g[1:2] * g[1:2])
                        + g[2:3] * g[2:3])

    p = posP_ref[...]
    h = jnp.maximum(_bdot(p, W1_ref[...]) + b1_ref[...], 0.0)
    h = jnp.maximum(_bdot(h, W2_ref[...]) + b2_ref[...], 0.0)
    feats_ref[...] = _bdot(h, W3_ref[...]) + b3_ref[...]

    asq = ((p[:, 0:1] * p[:, 0:1] + p[:, 1:2] * p[:, 1:2])
           + p[:, 2:3] * p[:, 2:3])
    p_bf = p.astype(jnp.bfloat16)
    run_min = jnp.full((BN, 1), jnp.inf, jnp.float32)
    run_arg = jnp.zeros((BN, 1), jnp.int32)
    for c in range(NC):
        sl = slice(c * CM, (c + 1) * CM)
        ag = jnp.dot(p_bf, gB_ref[:, sl],
                     preferred_element_type=jnp.float32)
        d2 = (asq - 2.0 * ag) + gsq_ref[:, sl]
        mn = jnp.min(d2, axis=1, keepdims=True)
        gi = jax.lax.broadcasted_iota(jnp.int32, (BN, CM), 1) + c * CM
        am = jnp.min(jnp.where(d2 == mn, gi, BIG), axis=1, keepdims=True)
        upd = mn < run_min
        run_arg = jnp.where(upd, am, run_arg)
        run_min = jnp.minimum(run_min, mn)
    idx_ref[...] = jnp.swapaxes(run_arg, 0, 1)[None]


def _reduce_kernel(feats_ref, idxb_ref, idxf_ref, Wn1_ref, bn1_ref, Wn2_ref,
                   bn2_ref, Wh1_ref, bh1_ref, Wh2_ref, bh2_ref, out_ref,
                   acc_ref, cnt_ref):
    b = pl.program_id(0)

    @pl.when(b == 0)
    def _init():
        acc_ref[...] = jnp.zeros_like(acc_ref)
        cnt_ref[...] = jnp.zeros_like(cnt_ref)

    idx_i = jnp.swapaxes(idxb_ref[0], 0, 1)
    gi = jax.lax.broadcasted_iota(jnp.int32, (BN, 1), 0) + b * BN
    gj = jax.lax.broadcasted_iota(jnp.int32, (1, BN), 1)
    dup = jnp.zeros((BN, 1), jnp.bool_)
    for jb in range(NB):
        cmp = (idxf_ref[jb] == idx_i) & ((gj + jb * BN) > gi)
        dup = dup | jnp.any(cmp, axis=1, keepdims=True)
    w = jnp.where(dup, 0.0, 1.0)

    z = jnp.maximum(_bdot(feats_ref[...], Wn1_ref[...]) + bn1_ref[...], 0.0)
    acc_ref[...] += jnp.sum(z * w, axis=0, keepdims=True)
    cnt_ref[...] += jnp.sum(w, axis=0, keepdims=True)

    @pl.when(b == NB - 1)
    def _final():
        nocc = cnt_ref[...]
        srel = acc_ref[...] + (M - nocc) * jnp.maximum(bn1_ref[...], 0.0)
        agg = _bdot(srel * (1.0 / M), Wn2_ref[...]) + bn2_ref[...]
        hh = jnp.maximum(_bdot(agg, Wh1_ref[...]) + bh1_ref[...], 0.0)
        prod = (hh.astype(jnp.bfloat16).astype(jnp.float32)
                * Wh2_ref[...].astype(jnp.bfloat16).astype(jnp.float32))
        hw = H // 2
        for k in range(4):
            s = jnp.sum(prod[:, k * hw:(k + 1) * hw], axis=1, keepdims=True)
            out_ref[:, k:k + 1] = s + bh2_ref[:, k:k + 1]


def kernel(atomic_positions, grid_points, W1, b1, W2, b2, W3, b3,
           Wn1, bn1, Wn2, bn2, Wh1, bh1, Wh2, bh2):
    f32 = jnp.float32
    posP = jnp.pad(atomic_positions, ((0, 0), (0, 5)))
    gridP = jnp.pad(grid_points.T, ((0, 5), (0, 0)))
    W1P = jnp.pad(W1, ((0, 5), (0, 0)))
    b1r = b1.reshape(1, -1)
    b2r = b2.reshape(1, -1)
    b3r = b3.reshape(1, -1)
    bn1r = bn1.reshape(1, -1)
    bn2r = bn2.reshape(1, -1)
    Wh1r = Wh1.transpose(1, 0, 2).reshape(H, 4 * (H // 2))
    bh1r = bh1.reshape(1, -1)
    Wh2r = Wh2[:, :, 0].reshape(1, -1)
    bh2r = bh2.reshape(1, -1)

    feats, idx = pl.pallas_call(
        _enc_argmin_kernel,
        grid=(NB,),
        in_specs=[
            pl.BlockSpec((BN, 8), lambda b: (b, 0)),
            pl.BlockSpec((8, M), lambda b: (0, 0)),
            pl.BlockSpec((8, H // 4), lambda b: (0, 0)),
            pl.BlockSpec((1, H // 4), lambda b: (0, 0)),
            pl.BlockSpec((H // 4, H // 2), lambda b: (0, 0)),
            pl.BlockSpec((1, H // 2), lambda b: (0, 0)),
            pl.BlockSpec((H // 2, H), lambda b: (0, 0)),
            pl.BlockSpec((1, H), lambda b: (0, 0)),
        ],
        out_specs=[
            pl.BlockSpec((BN, H), lambda b: (b, 0)),
            pl.BlockSpec((1, 1, BN), lambda b: (b, 0, 0)),
        ],
        out_shape=[
            jax.ShapeDtypeStruct((N, H), f32),
            jax.ShapeDtypeStruct((NB, 1, BN), jnp.int32),
        ],
        scratch_shapes=[pltpu.VMEM((8, M), jnp.bfloat16),
                        pltpu.VMEM((1, M), f32)],
    )(posP, gridP, W1P, b1r, W2, b2r, W3, b3r)

    out = pl.pallas_call(
        _reduce_kernel,
        grid=(NB,),
        in_specs=[
            pl.BlockSpec((BN, H), lambda b: (b, 0)),
            pl.BlockSpec((1, 1, BN), lambda b: (b, 0, 0)),
            pl.BlockSpec((NB, 1, BN), lambda b: (0, 0, 0)),
            pl.BlockSpec((H, H), lambda b: (0, 0)),
            pl.BlockSpec((1, H), lambda b: (0, 0)),
            pl.BlockSpec((H, H), lambda b: (0, 0)),
            pl.BlockSpec((1, H), lambda b: (0, 0)),
            pl.BlockSpec((H, 4 * (H // 2)), lambda b: (0, 0)),
            pl.BlockSpec((1, 4 * (H // 2)), lambda b: (0, 0)),
            pl.BlockSpec((1, 4 * (H // 2)), lambda b: (0, 0)),
            pl.BlockSpec((1, 4), lambda b: (0, 0)),
        ],
        out_specs=pl.BlockSpec((1, 4), lambda b: (0, 0)),
        out_shape=jax.ShapeDtypeStruct((1, 4), f32),
        scratch_shapes=[pltpu.VMEM((1, H), f32), pltpu.VMEM((1, 1), f32)],
    )(feats, idx, idx, Wn1, bn1r, Wn2, bn2r, Wh1r, bh1r, Wh2r, bh2r)

    return out.reshape(4)


# speedup vs baseline: 1.6664x; 1.1354x over previous
import jax
import jax.numpy as jnp
from jax.experimental import pallas as pl
from jax.experimental.pallas import tpu as pltpu

N = 4096
M = 8192
H = 256
BN = 512
NB = N // BN
CM = 2048
NC = M // CM
BIG = 2 ** 30


def _bdot(a, b):
    return jnp.dot(a.astype(jnp.bfloat16), b.astype(jnp.bfloat16),
                   preferred_element_type=jnp.float32)


def _enc_argmin_kernel(posP_ref, gridP_ref, W1_ref, b1_ref, W2_ref, b2_ref,
                       W3_ref, b3_ref, feats_ref, idx_ref, gB_ref, gsq_ref):
    b = pl.program_id(0)

    @pl.when(b == 0)
    def _build_grid_aug():
        g = gridP_ref[...]
        gB_ref[...] = g.astype(jnp.bfloat16)
        gsq_ref[...] = ((g[0:1] * g[0:1] + g[1:2] * g[1:2])
                        + g[2:3] * g[2:3])

    p = posP_ref[...]
    h = jnp.maximum(_bdot(p, W1_ref[...]) + b1_ref[...], 0.0)
    h = jnp.maximum(_bdot(h, W2_ref[...]) + b2_ref[...], 0.0)
    feats_ref[...] = _bdot(h, W3_ref[...]) + b3_ref[...]

    asq = ((p[:, 0:1] * p[:, 0:1] + p[:, 1:2] * p[:, 1:2])
           + p[:, 2:3] * p[:, 2:3])
    pm2_bf = (p * -2.0).astype(jnp.bfloat16)
    gi_f = jax.lax.broadcasted_iota(jnp.int32, (BN, CM), 1).astype(jnp.float32)
    run_min = jnp.full((BN, 1), jnp.inf, jnp.float32)
    run_arg = jnp.zeros((BN, 1), jnp.float32)
    for c in range(NC):
        sl = slice(c * CM, (c + 1) * CM)
        ag2 = jnp.dot(pm2_bf, gB_ref[:, sl],
                      preferred_element_type=jnp.float32)
        d2 = (asq + ag2) + gsq_ref[:, sl]
        mn = jnp.min(d2, axis=1, keepdims=True)
        am = jnp.min(jnp.where(d2 == mn, gi_f, 3e9), axis=1,
                     keepdims=True) + (c * CM)
        upd = mn < run_min
        run_arg = jnp.where(upd, am, run_arg)
        run_min = jnp.minimum(run_min, mn)
    idx_ref[...] = jnp.swapaxes(run_arg.astype(jnp.int32), 0, 1)[None]


def _reduce_kernel(feats_ref, idxb_ref, idxf_ref, Wn1_ref, bn1_ref, Wn2_ref,
                   bn2_ref, Wh1_ref, bh1_ref, Wh2_ref, bh2_ref, out_ref,
                   acc_ref, cnt_ref, dup_ref):
    b = pl.program_id(0)

    @pl.when(b == 0)
    def _init():
        acc_ref[...] = jnp.zeros_like(acc_ref)
        cnt_ref[...] = jnp.zeros_like(cnt_ref)

    idx_i = jnp.swapaxes(idxb_ref[0], 0, 1)
    dup_ref[...] = jnp.zeros_like(dup_ref)
    for jb in range(1, NB):
        @pl.when(jb > b)
        def _full_block():
            hit = jnp.any(idxf_ref[jb] == idx_i, axis=1, keepdims=True)
            dup_ref[...] = jnp.maximum(dup_ref[...], hit.astype(jnp.float32))

    tri = (jax.lax.broadcasted_iota(jnp.int32, (BN, BN), 1)
           > jax.lax.broadcasted_iota(jnp.int32, (BN, BN), 0))
    hit_d = jnp.any((idxb_ref[0] == idx_i) & tri, axis=1, keepdims=True)
    w = 1.0 - jnp.maximum(dup_ref[...], hit_d.astype(jnp.float32))

    z = jnp.maximum(_bdot(feats_ref[...], Wn1_ref[...]) + bn1_ref[...], 0.0)
    acc_ref[...] += jnp.sum(z * w, axis=0, keepdims=True)
    cnt_ref[...] += jnp.sum(w, axis=0, keepdims=True)

    @pl.when(b == NB - 1)
    def _final():
        nocc = cnt_ref[...]
        srel = acc_ref[...] + (M - nocc) * jnp.maximum(bn1_ref[...], 0.0)
        agg = _bdot(srel * (1.0 / M), Wn2_ref[...]) + bn2_ref[...]
        hh = jnp.maximum(_bdot(agg, Wh1_ref[...]) + bh1_ref[...], 0.0)
        prod = (hh.astype(jnp.bfloat16).astype(jnp.float32)
                * Wh2_ref[...].astype(jnp.bfloat16).astype(jnp.float32))
        hw = H // 2
        for k in range(4):
            s = jnp.sum(prod[:, k * hw:(k + 1) * hw], axis=1, keepdims=True)
            out_ref[:, k:k + 1] = s + bh2_ref[:, k:k + 1]


def kernel(atomic_positions, grid_points, W1, b1, W2, b2, W3, b3,
           Wn1, bn1, Wn2, bn2, Wh1, bh1, Wh2, bh2):
    f32 = jnp.float32
    posP = jnp.pad(atomic_positions, ((0, 0), (0, 5)))
    gridP = jnp.pad(grid_points.T, ((0, 5), (0, 0)))
    W1P = jnp.pad(W1, ((0, 5), (0, 0)))
    b1r = b1.reshape(1, -1)
    b2r = b2.reshape(1, -1)
    b3r = b3.reshape(1, -1)
    bn1r = bn1.reshape(1, -1)
    bn2r = bn2.reshape(1, -1)
    Wh1r = Wh1.transpose(1, 0, 2).reshape(H, 4 * (H // 2))
    bh1r = bh1.reshape(1, -1)
    Wh2r = Wh2[:, :, 0].reshape(1, -1)
    bh2r = bh2.reshape(1, -1)

    feats, idx = pl.pallas_call(
        _enc_argmin_kernel,
        grid=(NB,),
        in_specs=[
            pl.BlockSpec((BN, 8), lambda b: (b, 0)),
            pl.BlockSpec((8, M), lambda b: (0, 0)),
            pl.BlockSpec((8, H // 4), lambda b: (0, 0)),
            pl.BlockSpec((1, H // 4), lambda b: (0, 0)),
            pl.BlockSpec((H // 4, H // 2), lambda b: (0, 0)),
            pl.BlockSpec((1, H // 2), lambda b: (0, 0)),
            pl.BlockSpec((H // 2, H), lambda b: (0, 0)),
            pl.BlockSpec((1, H), lambda b: (0, 0)),
        ],
        out_specs=[
            pl.BlockSpec((BN, H), lambda b: (b, 0)),
            pl.BlockSpec((1, 1, BN), lambda b: (b, 0, 0)),
        ],
        out_shape=[
            jax.ShapeDtypeStruct((N, H), f32),
            jax.ShapeDtypeStruct((NB, 1, BN), jnp.int32),
        ],
        scratch_shapes=[pltpu.VMEM((8, M), jnp.bfloat16),
                        pltpu.VMEM((1, M), f32)],
    )(posP, gridP, W1P, b1r, W2, b2r, W3, b3r)

    out = pl.pallas_call(
        _reduce_kernel,
        grid=(NB,),
        in_specs=[
            pl.BlockSpec((BN, H), lambda b: (b, 0)),
            pl.BlockSpec((1, 1, BN), lambda b: (b, 0, 0)),
            pl.BlockSpec((NB, 1, BN), lambda b: (0, 0, 0)),
            pl.BlockSpec((H, H), lambda b: (0, 0)),
            pl.BlockSpec((1, H), lambda b: (0, 0)),
            pl.BlockSpec((H, H), lambda b: (0, 0)),
            pl.BlockSpec((1, H), lambda b: (0, 0)),
            pl.BlockSpec((H, 4 * (H // 2)), lambda b: (0, 0)),
            pl.BlockSpec((1, 4 * (H // 2)), lambda b: (0, 0)),
            pl.BlockSpec((1, 4 * (H // 2)), lambda b: (0, 0)),
            pl.BlockSpec((1, 4), lambda b: (0, 0)),
        ],
        out_specs=pl.BlockSpec((1, 4), lambda b: (0, 0)),
        out_shape=jax.ShapeDtypeStruct((1, 4), f32),
        scratch_shapes=[pltpu.VMEM((1, H), f32), pltpu.VMEM((1, 1), f32),
                        pltpu.VMEM((BN, 1), f32)],
    )(feats, idx, idx, Wn1, bn1r, Wn2, bn2r, Wh1r, bh1r, Wh2r, bh2r)

    return out.reshape(4)
